# speedup vs baseline: 1.0053x; 1.0053x over previous
import functools
import jax
import jax.numpy as jnp
from jax.experimental import pallas as pl
from jax.experimental.pallas import tpu as pltpu

N = 10000
NPAD = 10240
D = 128
B = 64
ROWS = 256


def _tc_mm_body(x_ref, w_ref, wa_ref, h_ref, a_ref):
    xb = jnp.maximum(x_ref[...], 0.0)
    h_ref[...] = jnp.dot(xb, w_ref[...], preferred_element_type=jnp.float32)
    a_ref[...] = jnp.dot(xb, wa_ref[...], preferred_element_type=jnp.float32)


def _tc_matmul(x, w, wa):
    grid = (NPAD // ROWS,)
    return pl.pallas_call(
        _tc_mm_body,
        grid=grid,
        in_specs=[
            pl.BlockSpec((ROWS, D), lambda i: (i, 0)),
            pl.BlockSpec((D, D), lambda i: (0, 0)),
            pl.BlockSpec((D, D), lambda i: (0, 0)),
        ],
        out_specs=[
            pl.BlockSpec((ROWS, D), lambda i: (i, 0)),
            pl.BlockSpec((ROWS, D), lambda i: (i, 0)),
        ],
        out_shape=[
            jax.ShapeDtypeStruct((NPAD, D), jnp.float32),
            jax.ShapeDtypeStruct((NPAD, D), jnp.float32),
        ],
    )(x, w, wa)


def _gat_edges(h, asv, adv, src, dst, b):
    e = jax.nn.leaky_relu(asv[src] + adv[dst], negative_slope=0.2)
    m = jax.ops.segment_max(e, dst, num_segments=N)
    m = jnp.where(jnp.isfinite(m), m, 0.0)
    ex = jnp.exp(e - m[dst])
    denom = jax.ops.segment_sum(ex, dst, num_segments=N)
    alpha = ex / (denom[dst] + 1e-16)
    out = jax.ops.segment_sum(h[src] * alpha[:, None], dst, num_segments=N)
    return out + b


def kernel(x, edge_index, batch, W1, a_src1, a_dst1, b1, W2, a_src2, a_dst2, b2, Wfc, bfc):
    loop = jnp.arange(N, dtype=edge_index.dtype)
    src = jnp.concatenate([edge_index[0], loop])
    dst = jnp.concatenate([edge_index[1], loop])

    xp = jnp.zeros((NPAD, D), jnp.float32).at[:N].set(x)

    wa1 = jnp.zeros((D, D), jnp.float32)
    wa1 = wa1.at[:, 0].set(W1 @ a_src1).at[:, 1].set(W1 @ a_dst1)
    h1, al1 = _tc_matmul(xp, W1, wa1)
    y1 = _gat_edges(h1[:N], al1[:N, 0], al1[:N, 1], src, dst, b1)

    wa2 = jnp.zeros((D, D), jnp.float32)
    wa2 = wa2.at[:, 0].set(W2 @ a_src2).at[:, 1].set(W2 @ a_dst2)
    y1p = jnp.zeros((NPAD, D), jnp.float32).at[:N].set(y1)
    h2, al2 = _tc_matmul(y1p, W2, wa2)
    y2 = _gat_edges(h2[:N], al2[:N, 0], al2[:N, 1], src, dst, b2)

    x3 = jax.nn.relu(y2)
    gmax = jax.ops.segment_max(x3, batch, num_segments=B)
    gmax = jnp.where(jnp.isfinite(gmax), gmax, 0.0)
    cnt = jax.ops.segment_sum(jnp.ones((N, 1), x3.dtype), batch, num_segments=B)
    gmean = jax.ops.segment_sum(x3, batch, num_segments=B) / jnp.maximum(cnt, 1.0)
    g = jnp.concatenate([gmax, gmean], axis=1)
    return g @ Wfc + bfc


# speedup vs baseline: 6.7276x; 6.6919x over previous
import dataclasses
import functools
import jax
import jax.numpy as jnp
from jax import lax
from jax.experimental import pallas as pl
from jax.experimental.pallas import tpu as pltpu
from jax.experimental.pallas import tpu_sc as plsc

N = 10000
NPAD = 10240
D = 128
B = 64
SEG = 72
NSC = 2
NTILE = 16
NW = NSC * NTILE
L = 16
EPAD = 344064
EPT = EPAD // NW
KB = 128
RPT = NPAD // NW
RPS = NPAD // NTILE
AH = NPAD // 2
ACC_R = 5248
DUMP = AH
EPT2 = EPAD // NTILE
NB2 = EPT2 // KB
CB = 24
NCH = NB2 // CB
ROWS = 256

_mesh = plsc.VectorSubcoreMesh(core_axis_name="c", subcore_axis_name="s")

_sc_params = pltpu.CompilerParams()
if "needs_layout_passes" in pltpu.CompilerParams.__dataclass_fields__:
    _sc_params = dataclasses.replace(_sc_params, needs_layout_passes=False)


def _f32(shape):
    return jax.ShapeDtypeStruct(shape, jnp.float32)


def _tc1_body(x_ref, w_ref, wa_ref, h_ref, a_ref):
    xb = jnp.maximum(x_ref[...], 0.0)
    h = jnp.dot(xb, w_ref[...], preferred_element_type=jnp.float32)
    h_ref[...] = h
    a_ref[...] = jnp.dot(h, wa_ref[...], preferred_element_type=jnp.float32)


def _tc1(x, w, wa):
    return pl.pallas_call(
        _tc1_body,
        grid=(NPAD // ROWS,),
        in_specs=[
            pl.BlockSpec((ROWS, D), lambda i: (i, 0)),
            pl.BlockSpec((D, D), lambda i: (0, 0)),
            pl.BlockSpec((D, D), lambda i: (0, 0)),
        ],
        out_specs=[
            pl.BlockSpec((ROWS, D), lambda i: (i, 0)),
            pl.BlockSpec((ROWS, D), lambda i: (i, 0)),
        ],
        out_shape=[_f32((NPAD, D)), _f32((NPAD, D))],
    )(x, w, wa)


def _tc2_body(p_ref, b_ref, w_ref, wa_ref, h_ref, a_ref):
    xb = jnp.maximum(p_ref[...] + b_ref[...], 0.0)
    h = jnp.dot(xb, w_ref[...], preferred_element_type=jnp.float32)
    h_ref[...] = h
    a_ref[...] = jnp.dot(h, wa_ref[...], preferred_element_type=jnp.float32)


def _tc2(p, b2d, w, wa):
    return pl.pallas_call(
        _tc2_body,
        grid=(NPAD // ROWS,),
        in_specs=[
            pl.BlockSpec((ROWS, D), lambda i: (i, 0)),
            pl.BlockSpec((1, D), lambda i: (0, 0)),
            pl.BlockSpec((D, D), lambda i: (0, 0)),
            pl.BlockSpec((D, D), lambda i: (0, 0)),
        ],
        out_specs=[
            pl.BlockSpec((ROWS, D), lambda i: (i, 0)),
            pl.BlockSpec((ROWS, D), lambda i: (i, 0)),
        ],
        out_shape=[_f32((NPAD, D)), _f32((NPAD, D))],
    )(p, b2d, w, wa)


def _tc3_body(p_ref, b_ref, o_ref):
    o_ref[...] = jnp.maximum(p_ref[...] + b_ref[...], 0.0)


def _tc3(p, b2d):
    return pl.pallas_call(
        _tc3_body,
        grid=(NPAD // ROWS,),
        in_specs=[
            pl.BlockSpec((ROWS, D), lambda i: (i, 0)),
            pl.BlockSpec((1, D), lambda i: (0, 0)),
        ],
        out_specs=pl.BlockSpec((ROWS, D), lambda i: (i, 0)),
        out_shape=_f32((NPAD, D)),
    )(p, b2d)


def _tce_body(e_ref, o_ref):
    o_ref[...] = jnp.exp(e_ref[...])


def _tce(e):
    nr = EPAD // D // 8
    return pl.pallas_call(
        _tce_body,
        grid=(8,),
        in_specs=[pl.BlockSpec((nr, D), lambda i: (i, 0))],
        out_specs=pl.BlockSpec((nr, D), lambda i: (i, 0)),
        out_shape=_f32((EPAD // D, D)),
    )(e.reshape(EPAD // D, D)).reshape(EPAD)


def _tcd_body(p_ref, o_ref):
    o_ref[...] = p_ref[0:NPAD // D, :] + p_ref[NPAD // D:2 * NPAD // D, :] + 1e-16


def _tcd(denomp):
    return pl.pallas_call(
        _tcd_body,
        in_specs=[pl.BlockSpec((2 * NPAD // D, D), lambda: (0, 0))],
        out_specs=pl.BlockSpec((NPAD // D, D), lambda: (0, 0)),
        out_shape=_f32((NPAD // D, D)),
    )(denomp.reshape(2 * NPAD // D, D))


def _tc4_body(maxp_ref, sump_ref, cntp_ref, wfca_ref, wfcb_ref, bfc_ref, o_ref):
    mx = maxp_ref[0:SEG, :]
    sm = sump_ref[0:SEG, :]
    cn = cntp_ref[0:SEG, :]
    for t in range(1, NW):
        lo, hi = t * SEG, (t + 1) * SEG
        mx = jnp.maximum(mx, maxp_ref[lo:hi, :])
        sm = sm + sump_ref[lo:hi, :]
        cn = cn + cntp_ref[lo:hi, :]
    mx = mx[0:B, :]
    sm = sm[0:B, :]
    cn = cn[0:B, :]
    mx = mx[0:B, :]
    sm = sm[0:B, :]
    cn = cn[0:B, :]
    mx = jnp.where(jnp.isfinite(mx), mx, 0.0)
    cnc = jnp.maximum(cn, 1.0)
    rc = 1.0 / cnc
    rc = rc * (2.0 - cnc * rc)
    gmean = sm * rc
    dot = (jnp.dot(mx, wfca_ref[...], preferred_element_type=jnp.float32)
           + jnp.dot(gmean, wfcb_ref[...], preferred_element_type=jnp.float32))
    o_ref[...] = dot + bfc_ref[...]


def _tc4(maxp, sump, cntp, wfca, wfcb, bfc2):
    full = pl.BlockSpec((NW * SEG, D), lambda: (0, 0))
    return pl.pallas_call(
        _tc4_body,
        in_specs=[full, full, full,
                  pl.BlockSpec((D, 1), lambda: (0, 0)),
                  pl.BlockSpec((D, 1), lambda: (0, 0)),
                  pl.BlockSpec((1, 1), lambda: (0, 0))],
        out_specs=pl.BlockSpec((B, 1), lambda: (0, 0)),
        out_shape=_f32((B, 1)),
    )(maxp, sump, cntp, wfca, wfcb, bfc2)


@functools.partial(
    pl.kernel,
    out_type=_f32((EPAD,)),
    mesh=_mesh,
    compiler_params=_sc_params,
    scratch_types=[
        pltpu.VMEM((NPAD,), jnp.float32),
        pltpu.VMEM((NPAD,), jnp.float32),
        pltpu.VMEM((EPT,), jnp.int32),
        pltpu.VMEM((EPT,), jnp.int32),
        pltpu.VMEM((EPT,), jnp.float32),
    ],
)
def _sc_logits(asv_hbm, adv_hbm, src_hbm, dst_hbm, e_hbm,
               as_buf, ad_buf, src_buf, dst_buf, e_buf):
    c = lax.axis_index("c")
    s = lax.axis_index("s")
    wid = c * NTILE + s
    base = wid * EPT

    pltpu.sync_copy(asv_hbm, as_buf)
    pltpu.sync_copy(adv_hbm, ad_buf)
    pltpu.sync_copy(src_hbm.at[pl.ds(base, EPT)], src_buf)
    pltpu.sync_copy(dst_hbm.at[pl.ds(base, EPT)], dst_buf)

    @pl.loop(0, EPT, step=L)
    def _(i):
        sv = src_buf[pl.ds(i, L)]
        dv = dst_buf[pl.ds(i, L)]
        e = plsc.load_gather(as_buf, [sv]) + plsc.load_gather(ad_buf, [dv])
        e_buf[pl.ds(i, L)] = jnp.where(e >= 0.0, e, 0.2 * e)

    pltpu.sync_copy(e_buf, e_hbm.at[pl.ds(base, EPT)])


@functools.partial(
    pl.kernel,
    out_type=[_f32((NSC * NPAD,)),
              _f32((NSC * EPAD,)),
              jax.ShapeDtypeStruct((NSC * EPAD,), jnp.int32),
              jax.ShapeDtypeStruct((NSC * EPAD,), jnp.int32)],
    mesh=_mesh,
    compiler_params=_sc_params,
    scratch_types=[
        pltpu.VMEM((EPT,), jnp.int32),
        pltpu.VMEM((EPT,), jnp.float32),
        pltpu.VMEM((EPT,), jnp.float32),
        pltpu.VMEM((EPT,), jnp.int32),
        pltpu.VMEM((NPAD,), jnp.float32),
        pltpu.VMEM_SHARED((NTILE * NPAD,), jnp.float32),
        pltpu.VMEM((RPS,), jnp.float32),
        pltpu.VMEM((RPS,), jnp.float32),
    ],
)
def _sc_pass1(ex_hbm, dst_hbm, denomp_hbm, exm_hbm,
              dloc_hbm, dlocb_hbm,
              dst_buf, ex_buf, exm_buf, dloc_buf,
              den_buf, slab, red_buf, tmp_buf):
    c = lax.axis_index("c")
    s = lax.axis_index("s")
    wid = c * NTILE + s
    base = wid * EPT

    pltpu.sync_copy(dst_hbm.at[pl.ds(base, EPT)], dst_buf)
    pltpu.sync_copy(ex_hbm.at[pl.ds(base, EPT)], ex_buf)

    zeros = jnp.zeros((L,), jnp.float32)
    lane = jnp.arange(L, dtype=jnp.int32)

    @pl.loop(0, NPAD, step=L)
    def _(i):
        den_buf[pl.ds(i, L)] = zeros

    @pl.loop(0, EPT, step=L)
    def _(i):
        dv = dst_buf[pl.ds(i, L)]
        ex = ex_buf[pl.ds(i, L)]
        for ln in range(L):
            plsc.addupdate_scatter(den_buf, [dv], ex, mask=lane == ln)

    for cc in range(NSC):
        lo = cc * AH

        @pl.loop(0, EPT, step=L)
        def _(i):
            dv = dst_buf[pl.ds(i, L)]
            ex = ex_buf[pl.ds(i, L)]
            own = (dv >= lo) & (dv < lo + AH)
            exm_buf[pl.ds(i, L)] = jnp.where(own, ex, 0.0)
            dloc_buf[pl.ds(i, L)] = jnp.where(own, dv - lo, DUMP)

        pltpu.sync_copy(exm_buf, exm_hbm.at[pl.ds(cc * EPAD + base, EPT)])
        pltpu.sync_copy(dloc_buf, dloc_hbm.at[pl.ds(cc * EPAD + base, EPT)])
        pltpu.sync_copy(dloc_buf, dlocb_hbm.at[pl.ds(cc * EPAD + base, EPT)])

    pltpu.sync_copy(den_buf, slab.at[pl.ds(s * NPAD, NPAD)])
    plsc.subcore_barrier()
    col = s * RPS
    pltpu.sync_copy(slab.at[pl.ds(col, RPS)], red_buf)
    for t in range(1, NTILE):
        pltpu.sync_copy(slab.at[pl.ds(t * NPAD + col, RPS)], tmp_buf)

        @pl.loop(0, RPS, step=L)
        def _(i):
            red_buf[pl.ds(i, L)] += tmp_buf[pl.ds(i, L)]

    pltpu.sync_copy(red_buf, denomp_hbm.at[pl.ds(c * NPAD + col, RPS)])


@functools.partial(
    pl.kernel,
    out_type=_f32((NPAD, D)),
    mesh=_mesh,
    compiler_params=_sc_params,
    scratch_types=[
        pltpu.VMEM((ACC_R,), jnp.float32),
        pltpu.VMEM((CB, KB), jnp.int32),
        pltpu.VMEM((CB, KB), jnp.int32),
        pltpu.VMEM((CB * KB,), jnp.int32),
        pltpu.VMEM((CB * KB,), jnp.float32),
        pltpu.VMEM((KB,), jnp.float32),
        pltpu.VMEM((KB, D), jnp.float32),
        pltpu.VMEM((8, D), jnp.float32),
        pltpu.VMEM_SHARED((ACC_R, D), jnp.float32),
    ],
)
def _sc_pass2(h_hbm, den_hbm, exm_hbm, src2_hbm, dloc2_hbm, dlocf_hbm,
              out_hbm,
              den_buf, src2, dloc2, dlocf, exf, albuf, rows, zbuf, acc):
    c = lax.axis_index("c")
    s = lax.axis_index("s")
    ebase = c * EPAD + s * EPT2

    pltpu.sync_copy(den_hbm.at[pl.ds(c * AH, AH)], den_buf.at[pl.ds(0, AH)])
    den_buf[pl.ds(DUMP, L)] = jnp.ones((L,), jnp.float32)

    zeros = jnp.zeros((L,), jnp.float32)
    for r in range(8):
        for q in range(D // L):
            zbuf[r, pl.ds(q * L, L)] = zeros
    zrow0 = s * (ACC_R // NTILE)
    for t in range(ACC_R // NTILE // 8):
        pltpu.sync_copy(zbuf, acc.at[pl.ds(zrow0 + t * 8, 8)])
    plsc.subcore_barrier()

    for ch in range(NCH):
        rbase = s * NB2 + ch * CB
        fbase = ebase + ch * CB * KB
        pltpu.sync_copy(src2_hbm.at[pl.ds(rbase, CB)], src2)
        pltpu.sync_copy(dloc2_hbm.at[pl.ds(c * (EPAD // KB) + rbase, CB)],
                        dloc2)
        pltpu.sync_copy(dlocf_hbm.at[pl.ds(fbase, CB * KB)], dlocf)
        pltpu.sync_copy(exm_hbm.at[pl.ds(fbase, CB * KB)], exf)

        @pl.loop(0, CB)
        def _(j):
            eb = j * KB
            for k in range(KB // L):
                dv = dlocf[pl.ds(eb + k * L, L)]
                dn = plsc.load_gather(den_buf, [dv])
                exv = exf[pl.ds(eb + k * L, L)]
                rc = 1.0 / dn
                rc = rc * (2.0 - dn * rc)
                albuf[pl.ds(k * L, L)] = exv * rc
            pltpu.sync_copy(h_hbm.at[src2.at[j]], rows)

            @pl.loop(0, KB)
            def _(r):
                av = plsc.load_gather(
                    albuf, [jnp.zeros((L,), jnp.int32) + r])
                for q in range(D // L):
                    rows[r, pl.ds(q * L, L)] = rows[r, pl.ds(q * L, L)] * av

            pltpu.sync_copy(rows, acc.at[dloc2.at[j]], add=True)

    plsc.subcore_barrier()
    rb0 = s * (AH // NTILE)
    for t in range(AH // NTILE // 64):
        pltpu.sync_copy(acc.at[pl.ds(rb0 + t * 64, 64)],
                        out_hbm.at[pl.ds(c * AH + rb0 + t * 64, 64)])


@functools.partial(
    pl.kernel,
    out_type=[_f32((NW * SEG * D,)),
              _f32((NW * SEG * D,)),
              _f32((NW * SEG * D,))],
    mesh=_mesh,
    compiler_params=_sc_params,
    scratch_types=[
        pltpu.VMEM((RPT * D,), jnp.float32),
        pltpu.VMEM((RPT + L,), jnp.int32),
        pltpu.VMEM((SEG * D,), jnp.float32),
        pltpu.VMEM((SEG * D,), jnp.float32),
        pltpu.VMEM((SEG * D,), jnp.float32),
    ],
)
def _sc_pool(x3_hbm, batch_hbm, maxp_hbm, sump_hbm, cntp_hbm,
             xb, bb, accm, accs, accc):
    c = lax.axis_index("c")
    s = lax.axis_index("s")
    wid = c * NTILE + s
    nrow0 = wid * RPT

    pltpu.sync_copy(x3_hbm.at[pl.ds(nrow0 * D, RPT * D)], xb)
    pltpu.sync_copy(batch_hbm.at[pl.ds(nrow0, RPT)], bb.at[pl.ds(0, RPT)])

    neg = jnp.full((L,), -jnp.inf, jnp.float32)
    zeros = jnp.zeros((L,), jnp.float32)
    nq = D // L

    @pl.loop(0, SEG * D, step=L)
    def _(i):
        accm[pl.ds(i, L)] = neg
        accs[pl.ds(i, L)] = zeros
        accc[pl.ds(i, L)] = zeros

    def _body(r, carry):
        prev = carry[0]
        cnt = carry[1]
        rm = carry[2:2 + nq]
        rs = carry[2 + nq:2 + 2 * nq]
        b = bb[pl.ds(r, L)][0]
        newseg = b != prev

        @pl.when(newseg & (prev >= 0))
        def _():
            for q in range(nq):
                off = prev * D + q * L
                accm[pl.ds(off, L)] = rm[q]
                accs[pl.ds(off, L)] = rs[q]
                accc[pl.ds(off, L)] = zeros + cnt

        xr = [xb[pl.ds(r * D + q * L, L)] for q in range(nq)]
        rm2 = tuple(jnp.where(newseg, xr[q], jnp.maximum(rm[q], xr[q]))
                    for q in range(nq))
        rs2 = tuple(jnp.where(newseg, xr[q], rs[q] + xr[q])
                    for q in range(nq))
        cnt2 = jnp.where(newseg, 1.0, cnt + 1.0)
        return (b, cnt2) + rm2 + rs2

    init = ((jnp.int32(-1), jnp.float32(0.0))
            + tuple(neg for _ in range(nq)) + tuple(zeros for _ in range(nq)))
    fin = jax.lax.fori_loop(0, RPT, _body, init)
    prev = fin[0]
    cnt = fin[1]
    for q in range(nq):
        off = prev * D + q * L
        accm[pl.ds(off, L)] = fin[2 + q]
        accs[pl.ds(off, L)] = fin[2 + nq + q]
        accc[pl.ds(off, L)] = zeros + cnt

    obase = wid * SEG * D
    pltpu.sync_copy(accm, maxp_hbm.at[pl.ds(obase, SEG * D)])
    pltpu.sync_copy(accs, sump_hbm.at[pl.ds(obase, SEG * D)])
    pltpu.sync_copy(accc, cntp_hbm.at[pl.ds(obase, SEG * D)])


def kernel(x, edge_index, batch, W1, a_src1, a_dst1, b1, W2, a_src2, a_dst2,
           b2, Wfc, bfc):
    E = edge_index.shape[1]
    npad_e = EPAD - (E + N)
    loop = jnp.arange(N, dtype=jnp.int32)
    src = jnp.concatenate(
        [edge_index[0].astype(jnp.int32), loop,
         jnp.zeros((npad_e,), jnp.int32)])
    dst = jnp.concatenate(
        [edge_index[1].astype(jnp.int32), loop,
         jnp.full((npad_e,), N, jnp.int32)])
    src2 = src.reshape(EPAD // KB, KB)

    xp = jnp.zeros((NPAD, D), jnp.float32).at[:N].set(x)
    batch_pad = jnp.concatenate(
        [batch.astype(jnp.int32), jnp.full((NPAD - N,), B, jnp.int32)])

    wa1 = jnp.zeros((D, D), jnp.float32)
    wa1 = wa1.at[:, 0].set(a_src1).at[:, 1].set(a_dst1)
    wa2 = jnp.zeros((D, D), jnp.float32)
    wa2 = wa2.at[:, 0].set(a_src2).at[:, 1].set(a_dst2)

    h1, al1 = _tc1(xp, W1, wa1)
    ex1 = _tce(_sc_logits(al1[:, 0], al1[:, 1], src, dst))
    denomp1, exm1, dloc1, dloc1b = _sc_pass1(ex1, dst)
    p1 = _sc_pass2(h1, _tcd(denomp1).reshape(NPAD), exm1, src2,
                   dloc1.reshape(NSC * EPAD // KB, KB), dloc1b)

    h2, al2 = _tc2(p1, b1.reshape(1, D), W2, wa2)
    ex2 = _tce(_sc_logits(al2[:, 0], al2[:, 1], src, dst))
    denomp2, exm2, dloc2, dloc2b = _sc_pass1(ex2, dst)
    p2 = _sc_pass2(h2, _tcd(denomp2).reshape(NPAD), exm2, src2,
                   dloc2.reshape(NSC * EPAD // KB, KB), dloc2b)

    x3 = _tc3(p2, b2.reshape(1, D))
    maxp, sump, cntp = _sc_pool(x3.reshape(NPAD * D), batch_pad)
    return _tc4(maxp.reshape(NW * SEG, D), sump.reshape(NW * SEG, D),
                cntp.reshape(NW * SEG, D), Wfc[:D].reshape(D, 1),
                Wfc[D:].reshape(D, 1), bfc.reshape(1, 1))
